# initial kernel scaffold (unmeasured)
import jax
import jax.numpy as jnp
from jax import lax
from jax.experimental import pallas as pl
from jax.experimental.pallas import tpu as pltpu

N_DEV = 8
B_LOC = 2
SQ = 512
SKV = 512
HQ_LOC = 8
DH = 64
D_MODEL = 768
D_CHUNK = HQ_LOC * DH


def kernel(x, Wq, K_ext, V_ext, Wo):
    my = lax.axis_index("i")
    Kb = lax.dynamic_slice_in_dim(K_ext, my * B_LOC, B_LOC, axis=0)
    Vb = lax.dynamic_slice_in_dim(V_ext, my * B_LOC, B_LOC, axis=0)

    def body(x_ref, wq_ref, k_ref, v_ref, wo_ref, out_ref,
             wq_comm, wo_comm, k_scr, v_scr, q_scr, ctx_scr, bias_scr,
             wq_send, wq_recv, wo_send, wo_recv, k_sems, v_sems):
        my = lax.axis_index("i")
        left = (my + N_DEV - 1) % N_DEV
        right = (my + 1) % N_DEV

        barrier = pltpu.get_barrier_semaphore()
        for nbr in (left, right):
            pl.semaphore_signal(barrier, inc=1, device_id=(nbr,),
                                device_id_type=pl.DeviceIdType.MESH)
        pl.semaphore_wait(barrier, 2)

        rq = (lax.broadcasted_iota(jnp.int32, (SQ, SKV), 0) // 64) % 4
        ck = (lax.broadcasted_iota(jnp.int32, (SQ, SKV), 1) // 64) % 4
        bias_scr[...] = jnp.where(rq == ck, 0.0, -1e9).astype(jnp.float32)

        wq_comm[0] = wq_ref[...]
        wo_comm[0] = wo_ref[...]

        def kv_copies(g, slot):
            kc = pltpu.make_async_copy(
                k_ref.at[:, :, pl.ds(g * HQ_LOC, HQ_LOC), :],
                k_scr.at[slot], k_sems.at[slot])
            vc = pltpu.make_async_copy(
                v_ref.at[:, :, pl.ds(g * HQ_LOC, HQ_LOC), :],
                v_scr.at[slot], v_sems.at[slot])
            return kc, vc

        pending = kv_copies(my, 0)
        pending[0].start()
        pending[1].start()

        for h in range(N_DEV):
            slot = h % 2
            nxt = (h + 1) % 2
            g = (my + N_DEV - h) % N_DEV

            rdmas = []
            if h < N_DEV - 1:
                for comm, ssem, rsem in ((wq_comm, wq_send, wq_recv),
                                         (wo_comm, wo_send, wo_recv)):
                    r = pltpu.make_async_remote_copy(
                        src_ref=comm.at[slot], dst_ref=comm.at[nxt],
                        send_sem=ssem.at[slot], recv_sem=rsem.at[nxt],
                        device_id=(right,),
                        device_id_type=pl.DeviceIdType.MESH)
                    r.start()
                    rdmas.append(r)
                nxt_copies = kv_copies((my + N_DEV - h - 1) % N_DEV, nxt)
                nxt_copies[0].start()
                nxt_copies[1].start()

            pending[0].wait()
            pending[1].wait()

            for b in range(B_LOC):
                q_scr[b] = jnp.dot(x_ref[b], wq_comm[slot],
                                   preferred_element_type=jnp.float32)

            def attn_body(idx, _, slot=slot):
                b = idx // HQ_LOC
                hh = idx % HQ_LOC
                q = q_scr[b, :, pl.ds(hh * DH, DH)]
                k = k_scr[slot, b, :, hh, :]
                s = lax.dot_general(q, k, (((1,), (1,)), ((), ())),
                                    preferred_element_type=jnp.float32)
                s = s * 0.125 + bias_scr[...]
                m = jnp.max(s, axis=1, keepdims=True)
                e = jnp.exp(s - m)
                w = e / jnp.sum(e, axis=1, keepdims=True)
                v = v_scr[slot, b, :, hh, :]
                ctx = lax.dot_general(w, v, (((1,), (0,)), ((), ())),
                                      preferred_element_type=jnp.float32)
                ctx_scr[b, :, pl.ds(hh * DH, DH)] = ctx
                return 0

            lax.fori_loop(0, B_LOC * HQ_LOC, attn_body, 0)

            for b in range(B_LOC):
                contrib = jnp.dot(ctx_scr[b], wo_comm[slot],
                                  preferred_element_type=jnp.float32)
                if h == 0:
                    out_ref[b] = contrib
                else:
                    out_ref[b] = out_ref[b] + contrib

            if h < N_DEV - 1:
                for r in rdmas:
                    r.wait()
                pending = nxt_copies

    return pl.pallas_call(
        body,
        out_shape=jax.ShapeDtypeStruct((B_LOC, SQ, D_MODEL), jnp.float32),
        in_specs=[
            pl.BlockSpec(memory_space=pltpu.VMEM),
            pl.BlockSpec(memory_space=pltpu.VMEM),
            pl.BlockSpec(memory_space=pltpu.ANY),
            pl.BlockSpec(memory_space=pltpu.ANY),
            pl.BlockSpec(memory_space=pltpu.VMEM),
        ],
        out_specs=pl.BlockSpec(memory_space=pltpu.VMEM),
        scratch_shapes=[
            pltpu.VMEM((2, D_MODEL, D_CHUNK), jnp.float32),
            pltpu.VMEM((2, D_CHUNK, D_MODEL), jnp.float32),
            pltpu.VMEM((2, B_LOC, SKV, HQ_LOC, DH), jnp.float32),
            pltpu.VMEM((2, B_LOC, SKV, HQ_LOC, DH), jnp.float32),
            pltpu.VMEM((B_LOC, SQ, D_CHUNK), jnp.float32),
            pltpu.VMEM((B_LOC, SQ, D_CHUNK), jnp.float32),
            pltpu.VMEM((SQ, SKV), jnp.float32),
            pltpu.SemaphoreType.DMA((2,)),
            pltpu.SemaphoreType.DMA((2,)),
            pltpu.SemaphoreType.DMA((2,)),
            pltpu.SemaphoreType.DMA((2,)),
            pltpu.SemaphoreType.DMA((2,)),
            pltpu.SemaphoreType.DMA((2,)),
        ],
        compiler_params=pltpu.CompilerParams(collective_id=0),
    )(x, Wq, Kb, Vb, Wo)


# baseline (device time: 354029 ns/iter reference)
import jax
import jax.numpy as jnp
from jax import lax
from jax.experimental import pallas as pl
from jax.experimental.pallas import tpu as pltpu

N_DEV = 8
B_LOC = 2
SQ = 512
SKV = 512
HQ_LOC = 8
DH = 64
D_MODEL = 768
D_CHUNK = HQ_LOC * DH


def kernel(x, Wq, K_ext, V_ext, Wo):
    my = lax.axis_index("i")
    Kb = lax.dynamic_slice_in_dim(K_ext, my * B_LOC, B_LOC, axis=0)
    Vb = lax.dynamic_slice_in_dim(V_ext, my * B_LOC, B_LOC, axis=0)

    def body(x_ref, wq_ref, k_ref, v_ref, wo_ref, out_ref,
             wq_comm, wo_comm, k_scr, v_scr, q_scr, ctx_scr, bias_scr,
             wq_send, wq_recv, wo_send, wo_recv, k_sems, v_sems):
        my = lax.axis_index("i")
        left = (my + N_DEV - 1) % N_DEV
        right = (my + 1) % N_DEV

        barrier = pltpu.get_barrier_semaphore()
        for nbr in (left, right):
            pl.semaphore_signal(barrier, inc=1, device_id=(nbr,),
                                device_id_type=pl.DeviceIdType.MESH)
        pl.semaphore_wait(barrier, 2)

        rq = (lax.broadcasted_iota(jnp.int32, (SQ, SKV), 0) // 64) % 4
        ck = (lax.broadcasted_iota(jnp.int32, (SQ, SKV), 1) // 64) % 4
        bias_scr[...] = jnp.where(rq == ck, 0.0, -1e9).astype(jnp.float32)

        wq_comm[0] = wq_ref[...]
        wo_comm[0] = wo_ref[...]

        def kv_copies(g, slot):
            copies = []
            for hh in range(HQ_LOC):
                copies.append(pltpu.make_async_copy(
                    k_ref.at[:, :, g * HQ_LOC + hh, :],
                    k_scr.at[slot, :, hh], k_sems.at[slot]))
                copies.append(pltpu.make_async_copy(
                    v_ref.at[:, :, g * HQ_LOC + hh, :],
                    v_scr.at[slot, :, hh], v_sems.at[slot]))
            return copies

        pending = kv_copies(my, 0)
        for c in pending:
            c.start()

        for h in range(N_DEV):
            slot = h % 2
            nxt = (h + 1) % 2
            g = (my + N_DEV - h) % N_DEV

            rdmas = []
            if h < N_DEV - 1:
                for comm, ssem, rsem in ((wq_comm, wq_send, wq_recv),
                                         (wo_comm, wo_send, wo_recv)):
                    r = pltpu.make_async_remote_copy(
                        src_ref=comm.at[slot], dst_ref=comm.at[nxt],
                        send_sem=ssem.at[slot], recv_sem=rsem.at[nxt],
                        device_id=(right,),
                        device_id_type=pl.DeviceIdType.MESH)
                    r.start()
                    rdmas.append(r)
                nxt_copies = kv_copies((my + N_DEV - h - 1) % N_DEV, nxt)
                for c in nxt_copies:
                    c.start()

            for c in pending:
                c.wait()

            for b in range(B_LOC):
                qfull = jnp.dot(x_ref[b], wq_comm[slot],
                                preferred_element_type=jnp.float32)
                for hh in range(HQ_LOC):
                    q_scr[b, hh] = qfull[:, hh * DH:(hh + 1) * DH]

            def attn_body(idx, _, slot=slot):
                b = idx // HQ_LOC
                hh = idx % HQ_LOC
                q = q_scr[b, hh]
                k = k_scr[slot, b, hh]
                s = lax.dot_general(q, k, (((1,), (1,)), ((), ())),
                                    preferred_element_type=jnp.float32)
                s = s * 0.125 + bias_scr[...]
                m = jnp.max(s, axis=1, keepdims=True)
                e = jnp.exp(s - m)
                w = e / jnp.sum(e, axis=1, keepdims=True)
                v = v_scr[slot, b, hh]
                ctx = lax.dot_general(w, v, (((1,), (0,)), ((), ())),
                                      preferred_element_type=jnp.float32)
                ctx_scr[b, hh] = ctx
                return 0

            lax.fori_loop(0, B_LOC * HQ_LOC, attn_body, 0)

            for b in range(B_LOC):
                ctx2d = jnp.concatenate(
                    [ctx_scr[b, hh] for hh in range(HQ_LOC)], axis=1)
                contrib = jnp.dot(ctx2d, wo_comm[slot],
                                  preferred_element_type=jnp.float32)
                if h == 0:
                    out_ref[b] = contrib
                else:
                    out_ref[b] = out_ref[b] + contrib

            if h < N_DEV - 1:
                for r in rdmas:
                    r.wait()
                pending = nxt_copies

    return pl.pallas_call(
        body,
        out_shape=jax.ShapeDtypeStruct((B_LOC, SQ, D_MODEL), jnp.float32),
        in_specs=[
            pl.BlockSpec(memory_space=pltpu.VMEM),
            pl.BlockSpec(memory_space=pltpu.VMEM),
            pl.BlockSpec(memory_space=pl.ANY),
            pl.BlockSpec(memory_space=pl.ANY),
            pl.BlockSpec(memory_space=pltpu.VMEM),
        ],
        out_specs=pl.BlockSpec(memory_space=pltpu.VMEM),
        scratch_shapes=[
            pltpu.VMEM((2, D_MODEL, D_CHUNK), jnp.float32),
            pltpu.VMEM((2, D_CHUNK, D_MODEL), jnp.float32),
            pltpu.VMEM((2, B_LOC, HQ_LOC, SKV, DH), jnp.float32),
            pltpu.VMEM((2, B_LOC, HQ_LOC, SKV, DH), jnp.float32),
            pltpu.VMEM((B_LOC, HQ_LOC, SQ, DH), jnp.float32),
            pltpu.VMEM((B_LOC, HQ_LOC, SQ, DH), jnp.float32),
            pltpu.VMEM((SQ, SKV), jnp.float32),
            pltpu.SemaphoreType.DMA((2,)),
            pltpu.SemaphoreType.DMA((2,)),
            pltpu.SemaphoreType.DMA((2,)),
            pltpu.SemaphoreType.DMA((2,)),
            pltpu.SemaphoreType.DMA((2,)),
            pltpu.SemaphoreType.DMA((2,)),
        ],
        compiler_params=pltpu.CompilerParams(
            collective_id=0, vmem_limit_bytes=50 * 1024 * 1024),
    )(x, Wq, Kb, Vb, Wo)
